# baseline (speedup 1.0000x reference)
import functools
import math

import jax
import jax.numpy as jnp
from jax import lax
from jax.experimental import pallas as pl
from jax.experimental.pallas import tpu as pltpu
from jax.experimental.pallas import tpu_sc as plsc

_BATCH = 16384
_DIM = 32
_PACK = 4
_PDIM = _DIM * _PACK
_NW = 32
_B_PER_W = _BATCH // _NW
_CHUNK = 128
_NCHUNK = _B_PER_W // _CHUNK
_LBLK = 4096


def _pack_body(in_ref, out_ref):
    x = in_ref[...]
    x3 = x.reshape(_DIM, _LBLK // _PACK, _PACK)
    y = jnp.transpose(x3, (1, 2, 0))
    out_ref[...] = y.reshape(_LBLK // _PACK, _PDIM)


def _tc_pack(tT):
    nr = tT.shape[1]
    grid = math.ceil(nr / _LBLK)
    return pl.pallas_call(
        _pack_body,
        grid=(grid,),
        in_specs=[pl.BlockSpec((_DIM, _LBLK), lambda i: (0, i))],
        out_specs=pl.BlockSpec((_LBLK // _PACK, _PDIM), lambda i: (i, 0)),
        out_shape=jax.ShapeDtypeStruct((nr // _PACK, _PDIM), jnp.float32),
    )(tT)


def _sc_gather(idx2d, Wu_p, rvh_p):
    mesh = plsc.VectorSubcoreMesh(core_axis_name="c", subcore_axis_name="s")

    @functools.partial(
        pl.kernel,
        out_type=(
            jax.ShapeDtypeStruct((_BATCH, _PDIM), jnp.float32),
            jax.ShapeDtypeStruct((_BATCH, _PDIM), jnp.float32),
        ),
        mesh=mesh,
        scratch_types=[
            pltpu.VMEM((_NCHUNK, _CHUNK), jnp.int32),
            pltpu.VMEM((_B_PER_W, _PDIM), jnp.float32),
            pltpu.SemaphoreType.DMA,
        ],
    )
    def gather_kernel(idx_hbm, wu_hbm, rvh_hbm, ru_out, rv_out,
                      idx_v, rows_v, sem):
        wid = lax.axis_index("s") * 2 + lax.axis_index("c")
        base = wid * _B_PER_W
        pltpu.sync_copy(idx_hbm.at[pl.ds(wid * _NCHUNK, _NCHUNK)], idx_v)
        for tbl, out in ((wu_hbm, ru_out), (rvh_hbm, rv_out)):
            copies = []
            for j in range(_NCHUNK):
                copies.append(pltpu.async_copy(
                    tbl.at[idx_v.at[j]],
                    rows_v.at[pl.ds(j * _CHUNK, _CHUNK)], sem))
            for c in copies:
                c.wait()
            pltpu.sync_copy(rows_v, out.at[pl.ds(base, _B_PER_W)])

    return gather_kernel(idx2d, Wu_p, rvh_p)


def _artanh(x):
    return 0.5 * jnp.log((1.0 + x) / (1.0 - x))


def _rownorm(x):
    return jnp.sqrt(jnp.sum(x * x, axis=-1, keepdims=True))


def _proj_rows(e):
    n = _rownorm(e)
    return jnp.where(n >= 1.0, e / (n - 1e-05), e)


def _p_sum(x, y):
    sqxnorm = jnp.sum(x * x, axis=-1, keepdims=True)
    sqynorm = jnp.sum(y * y, axis=-1, keepdims=True)
    dotxy = jnp.sum(x * y, axis=-1, keepdims=True)
    numerator = (1.0 + 2.0 * dotxy + sqynorm) * x + (1.0 - sqxnorm) * y
    denominator = 1.0 + 2.0 * dotxy + sqxnorm * sqynorm
    return numerator / denominator


def _select_subrow(packed, sub):
    out = jnp.zeros((packed.shape[0], _DIM), packed.dtype)
    for k in range(_PACK):
        sel = (sub == k).astype(packed.dtype)
        out = out + sel * packed[:, k * _DIM:(k + 1) * _DIM]
    return out


def _math_body(u_ref, v_ref, rup_ref, rvp_ref, sub_ref, out_ref):
    sub = sub_ref[...]
    Ru = _select_subrow(rup_ref[...], sub)
    rv_raw = _select_subrow(rvp_ref[...], sub)
    u = _proj_rows(u_ref[...])
    v = _proj_rows(v_ref[...])
    rv = _proj_rows(rv_raw)
    un = jnp.clip(_rownorm(u), 1e-10, 1.0 - 1e-05)
    u_e = _artanh(un) / un * u
    u_W = u_e * Ru
    wn = jnp.maximum(_rownorm(u_W), 1e-10)
    u_m = jnp.tanh(wn) / wn * u_W
    v_m = _p_sum(v, rv)
    u_m = _proj_rows(u_m)
    v_m = _proj_rows(v_m)
    diff = _p_sum(-u_m, v_m)
    diff_norm = jnp.clip(_rownorm(diff), 1e-10, 1.0 - 1e-05)
    sqdist = (2.0 * _artanh(diff_norm)) ** 2
    out_ref[...] = -sqdist


def _tc_math(u_emb, v_emb, Ru_p, Rv_p, sub, block_rows=2048):
    grid = _BATCH // block_rows
    row_spec = pl.BlockSpec((block_rows, _DIM), lambda i: (i, 0))
    packed_spec = pl.BlockSpec((block_rows, _PDIM), lambda i: (i, 0))
    sub_spec = pl.BlockSpec((block_rows, 1), lambda i: (i, 0))
    return pl.pallas_call(
        _math_body,
        grid=(grid,),
        in_specs=[row_spec, row_spec, packed_spec, packed_spec, sub_spec],
        out_specs=pl.BlockSpec((block_rows, 1), lambda i: (i, 0)),
        out_shape=jax.ShapeDtypeStruct((_BATCH, 1), jnp.float32),
    )(u_emb, v_emb, Ru_p, Rv_p, sub)


def kernel(u_emb, r_idx, v_emb, Wu, rvh):
    Wu_p = _tc_pack(Wu.T)
    rvh_p = _tc_pack(rvh.T)
    idx2d = (r_idx // _PACK).reshape(_NW * _NCHUNK, _CHUNK)
    sub = (r_idx % _PACK).reshape(_BATCH, 1)
    Ru_p, Rv_p = _sc_gather(idx2d, Wu_p, rvh_p)
    score = _tc_math(u_emb, v_emb, Ru_p, Rv_p, sub)
    return score.reshape(_BATCH)


# speedup vs baseline: 7.8780x; 7.8780x over previous
import functools

import jax
import jax.numpy as jnp
from jax import lax
from jax.experimental import pallas as pl
from jax.experimental.pallas import tpu as pltpu
from jax.experimental.pallas import tpu_sc as plsc

_BATCH = 16384
_DIM = 32
_PACK = 4
_PDIM = _DIM * _PACK
_NW = 32
_B_PER_W = _BATCH // _NW
_CHUNK = 128
_NCHUNK = _B_PER_W // _CHUNK
_LP = 2048
_GP = 123
_Q = _LP * _GP


def _pack_body(a0, a1, a2, a3, out_ref):
    for a, ref in enumerate((a0, a1, a2, a3)):
        out_ref[:, 32 * a:32 * (a + 1)] = ref[...].T


def _tc_pack(tT):
    max_blk = (tT.shape[1] - 1) // _LP
    specs = [
        pl.BlockSpec(
            (_DIM, _LP),
            (lambda a: (lambda i, a=a: (0, jnp.minimum(a * _GP + i,
                                                       max_blk))))(a))
        for a in range(_PACK)
    ]
    return pl.pallas_call(
        _pack_body,
        grid=(_GP,),
        in_specs=specs,
        out_specs=pl.BlockSpec((_LP, _PDIM), lambda i: (i, 0)),
        out_shape=jax.ShapeDtypeStruct((_Q, _PDIM), jnp.float32),
    )(tT, tT, tT, tT)


def _sc_gather(idx2d, Wu_p, rvh_p):
    mesh = plsc.VectorSubcoreMesh(core_axis_name="c", subcore_axis_name="s")

    @functools.partial(
        pl.kernel,
        out_type=(
            jax.ShapeDtypeStruct((_BATCH, _PDIM), jnp.float32),
            jax.ShapeDtypeStruct((_BATCH, _PDIM), jnp.float32),
        ),
        mesh=mesh,
        scratch_types=[
            pltpu.VMEM((_NCHUNK, _CHUNK), jnp.int32),
            pltpu.VMEM((_B_PER_W, _PDIM), jnp.float32),
            pltpu.SemaphoreType.DMA,
        ],
    )
    def gather_kernel(idx_hbm, wu_hbm, rvh_hbm, ru_out, rv_out,
                      idx_v, rows_v, sem):
        wid = lax.axis_index("s") * 2 + lax.axis_index("c")
        base = wid * _B_PER_W
        pltpu.sync_copy(idx_hbm.at[pl.ds(wid * _NCHUNK, _NCHUNK)], idx_v)
        for tbl, out in ((wu_hbm, ru_out), (rvh_hbm, rv_out)):
            copies = []
            for j in range(_NCHUNK):
                copies.append(pltpu.async_copy(
                    tbl.at[idx_v.at[j]],
                    rows_v.at[pl.ds(j * _CHUNK, _CHUNK)], sem))
            for c in copies:
                c.wait()
            pltpu.sync_copy(rows_v, out.at[pl.ds(base, _B_PER_W)])

    return gather_kernel(idx2d, Wu_p, rvh_p)


def _artanh(x):
    return 0.5 * jnp.log((1.0 + x) / (1.0 - x))


def _rownorm(x):
    return jnp.sqrt(jnp.sum(x * x, axis=-1, keepdims=True))


def _proj_rows(e):
    n = _rownorm(e)
    return jnp.where(n >= 1.0, e / (n - 1e-05), e)


def _p_sum(x, y):
    sqxnorm = jnp.sum(x * x, axis=-1, keepdims=True)
    sqynorm = jnp.sum(y * y, axis=-1, keepdims=True)
    dotxy = jnp.sum(x * y, axis=-1, keepdims=True)
    numerator = (1.0 + 2.0 * dotxy + sqynorm) * x + (1.0 - sqxnorm) * y
    denominator = 1.0 + 2.0 * dotxy + sqxnorm * sqynorm
    return numerator / denominator


def _select_subrow(packed, sub):
    out = jnp.zeros((packed.shape[0], _DIM), packed.dtype)
    for k in range(_PACK):
        out = jnp.where(sub == k, packed[:, k * _DIM:(k + 1) * _DIM], out)
    return out


def _math_body(u_ref, v_ref, rup_ref, rvp_ref, sub_ref, out_ref):
    sub = sub_ref[...]
    Ru = _select_subrow(rup_ref[...], sub)
    rv_raw = _select_subrow(rvp_ref[...], sub)
    u = _proj_rows(u_ref[...])
    v = _proj_rows(v_ref[...])
    rv = _proj_rows(rv_raw)
    un = jnp.clip(_rownorm(u), 1e-10, 1.0 - 1e-05)
    u_e = _artanh(un) / un * u
    u_W = u_e * Ru
    wn = jnp.maximum(_rownorm(u_W), 1e-10)
    u_m = jnp.tanh(wn) / wn * u_W
    v_m = _p_sum(v, rv)
    u_m = _proj_rows(u_m)
    v_m = _proj_rows(v_m)
    diff = _p_sum(-u_m, v_m)
    diff_norm = jnp.clip(_rownorm(diff), 1e-10, 1.0 - 1e-05)
    sqdist = (2.0 * _artanh(diff_norm)) ** 2
    out_ref[...] = -sqdist


def _tc_math(u_emb, v_emb, Ru_p, Rv_p, sub, block_rows=2048):
    grid = _BATCH // block_rows
    row_spec = pl.BlockSpec((block_rows, _DIM), lambda i: (i, 0))
    packed_spec = pl.BlockSpec((block_rows, _PDIM), lambda i: (i, 0))
    sub_spec = pl.BlockSpec((block_rows, 1), lambda i: (i, 0))
    return pl.pallas_call(
        _math_body,
        grid=(grid,),
        in_specs=[row_spec, row_spec, packed_spec, packed_spec, sub_spec],
        out_specs=pl.BlockSpec((block_rows, 1), lambda i: (i, 0)),
        out_shape=jax.ShapeDtypeStruct((_BATCH, 1), jnp.float32),
    )(u_emb, v_emb, Ru_p, Rv_p, sub)


def kernel(u_emb, r_idx, v_emb, Wu, rvh):
    Wu_p = _tc_pack(Wu.T)
    rvh_p = _tc_pack(rvh.T)
    idx2d = (r_idx % _Q).reshape(_NW * _NCHUNK, _CHUNK)
    sub = (r_idx // _Q).reshape(_BATCH, 1)
    Ru_p, Rv_p = _sc_gather(idx2d, Wu_p, rvh_p)
    score = _tc_math(u_emb, v_emb, Ru_p, Rv_p, sub)
    return score.reshape(_BATCH)


# speedup vs baseline: 13.4760x; 1.7106x over previous
import functools

import jax
import jax.numpy as jnp
from jax import lax
from jax.experimental import pallas as pl
from jax.experimental.pallas import tpu as pltpu
from jax.experimental.pallas import tpu_sc as plsc

_BATCH = 16384
_DIM = 32
_PACK = 4
_PDIM = _DIM * _PACK
_NW = 32
_B_PER_W = _BATCH // _NW
_CHUNK = 128
_NCHUNK = _B_PER_W // _CHUNK
_LP = 4096
_GP = 62
_Q = _LP * _GP


def _eye(n):
    r = jax.lax.broadcasted_iota(jnp.int32, (n, n), 0)
    c = jax.lax.broadcasted_iota(jnp.int32, (n, n), 1)
    return (r == c).astype(jnp.float32)


def _pack_body(a0, a1, a2, a3, out_ref):
    r = jax.lax.broadcasted_iota(jnp.int32, (_DIM, _PDIM), 0)
    c = jax.lax.broadcasted_iota(jnp.int32, (_DIM, _PDIM), 1)
    acc = None
    for a, ref in enumerate((a0, a1, a2, a3)):
        sel = (r == c - _DIM * a).astype(jnp.float32)
        y = jax.lax.dot_general(ref[...], sel, (((0,), (0,)), ((), ())),
                                preferred_element_type=jnp.float32)
        acc = y if acc is None else acc + y
    out_ref[...] = acc


def _tc_pack(tT):
    max_blk = (tT.shape[1] - 1) // _LP
    specs = [
        pl.BlockSpec(
            (_DIM, _LP),
            (lambda a: (lambda i, a=a: (0, jnp.minimum(a * _GP + i,
                                                       max_blk))))(a))
        for a in range(_PACK)
    ]
    return pl.pallas_call(
        _pack_body,
        grid=(_GP,),
        in_specs=specs,
        out_specs=pl.BlockSpec((_LP, _PDIM), lambda i: (i, 0)),
        out_shape=jax.ShapeDtypeStruct((_Q, _PDIM), jnp.float32),
    )(tT, tT, tT, tT)


def _sc_gather(idx2d, Wu_p, rvh_p):
    mesh = plsc.VectorSubcoreMesh(core_axis_name="c", subcore_axis_name="s")

    @functools.partial(
        pl.kernel,
        out_type=(
            jax.ShapeDtypeStruct((_BATCH, _PDIM), jnp.float32),
            jax.ShapeDtypeStruct((_BATCH, _PDIM), jnp.float32),
        ),
        mesh=mesh,
        scratch_types=[
            pltpu.VMEM((_NCHUNK, _CHUNK), jnp.int32),
            pltpu.VMEM((_B_PER_W, _PDIM), jnp.float32),
            pltpu.SemaphoreType.DMA,
        ],
    )
    def gather_kernel(idx_hbm, wu_hbm, rvh_hbm, ru_out, rv_out,
                      idx_v, rows_v, sem):
        wid = lax.axis_index("s") * 2 + lax.axis_index("c")
        base = wid * _B_PER_W
        pltpu.sync_copy(idx_hbm.at[pl.ds(wid * _NCHUNK, _NCHUNK)], idx_v)
        for tbl, out in ((wu_hbm, ru_out), (rvh_hbm, rv_out)):
            copies = []
            for j in range(_NCHUNK):
                copies.append(pltpu.async_copy(
                    tbl.at[idx_v.at[j]],
                    rows_v.at[pl.ds(j * _CHUNK, _CHUNK)], sem))
            for c in copies:
                c.wait()
            pltpu.sync_copy(rows_v, out.at[pl.ds(base, _B_PER_W)])

    return gather_kernel(idx2d, Wu_p, rvh_p)


def _artanh(x):
    return 0.5 * jnp.log((1.0 + x) / (1.0 - x))


def _colnorm(x):
    return jnp.sqrt(jnp.sum(x * x, axis=0, keepdims=True))


def _proj_cols(e):
    n = _colnorm(e)
    return jnp.where(n >= 1.0, e / (n - 1e-05), e)


def _p_sum_cols(x, y):
    sqxnorm = jnp.sum(x * x, axis=0, keepdims=True)
    sqynorm = jnp.sum(y * y, axis=0, keepdims=True)
    dotxy = jnp.sum(x * y, axis=0, keepdims=True)
    numerator = (1.0 + 2.0 * dotxy + sqynorm) * x + (1.0 - sqxnorm) * y
    denominator = 1.0 + 2.0 * dotxy + sqxnorm * sqynorm
    return numerator / denominator


def _select_subrow(packed, sub):
    out = jnp.zeros((packed.shape[0], _DIM), packed.dtype)
    for k in range(_PACK):
        out = jnp.where(sub == k, packed[:, k * _DIM:(k + 1) * _DIM], out)
    return out


def _math_body(uT_ref, vT_ref, rup_ref, rvp_ref, sub_ref, out_ref):
    sub = sub_ref[...]
    eye = _eye(_DIM)
    Ru_r = _select_subrow(rup_ref[...], sub)
    rv_r = _select_subrow(rvp_ref[...], sub)
    Ru = jax.lax.dot_general(eye, Ru_r, (((0,), (1,)), ((), ())),
                             preferred_element_type=jnp.float32)
    rv_raw = jax.lax.dot_general(eye, rv_r, (((0,), (1,)), ((), ())),
                                 preferred_element_type=jnp.float32)
    u = _proj_cols(uT_ref[...])
    v = _proj_cols(vT_ref[...])
    rv = _proj_cols(rv_raw)
    un = jnp.clip(_colnorm(u), 1e-10, 1.0 - 1e-05)
    u_e = _artanh(un) / un * u
    u_W = u_e * Ru
    wn = jnp.maximum(_colnorm(u_W), 1e-10)
    u_m = jnp.tanh(wn) / wn * u_W
    v_m = _p_sum_cols(v, rv)
    u_m = _proj_cols(u_m)
    v_m = _proj_cols(v_m)
    diff = _p_sum_cols(-u_m, v_m)
    diff_norm = jnp.clip(_colnorm(diff), 1e-10, 1.0 - 1e-05)
    sqdist = (2.0 * _artanh(diff_norm)) ** 2
    out_ref[...] = -sqdist


def _tc_math(uT, vT, Ru_p, Rv_p, sub, block_rows=2048):
    grid = _BATCH // block_rows
    t_spec = pl.BlockSpec((_DIM, block_rows), lambda i: (0, i))
    packed_spec = pl.BlockSpec((block_rows, _PDIM), lambda i: (i, 0))
    sub_spec = pl.BlockSpec((block_rows, 1), lambda i: (i, 0))
    return pl.pallas_call(
        _math_body,
        grid=(grid,),
        in_specs=[t_spec, t_spec, packed_spec, packed_spec, sub_spec],
        out_specs=pl.BlockSpec((1, block_rows), lambda i: (0, i)),
        out_shape=jax.ShapeDtypeStruct((1, _BATCH), jnp.float32),
    )(uT, vT, Ru_p, Rv_p, sub)


def kernel(u_emb, r_idx, v_emb, Wu, rvh):
    Wu_p = _tc_pack(Wu.T)
    rvh_p = _tc_pack(rvh.T)
    idx2d = (r_idx % _Q).reshape(_NW * _NCHUNK, _CHUNK)
    sub = (r_idx // _Q).reshape(_BATCH, 1)
    Ru_p, Rv_p = _sc_gather(idx2d, Wu_p, rvh_p)
    score = _tc_math(u_emb.T, v_emb.T, Ru_p, Rv_p, sub)
    return score.reshape(_BATCH)


# speedup vs baseline: 14.3603x; 1.0656x over previous
import functools

import jax
import jax.numpy as jnp
from jax import lax
from jax.experimental import pallas as pl
from jax.experimental.pallas import tpu as pltpu
from jax.experimental.pallas import tpu_sc as plsc

_BATCH = 16384
_DIM = 32
_PACK = 4
_PDIM = _DIM * _PACK
_NW = 32
_B_PER_W = _BATCH // _NW
_CHUNK = 128
_NCHUNK = _B_PER_W // _CHUNK
_LP = 4096
_GP = 62
_Q = _LP * _GP


def _eye(n):
    r = jax.lax.broadcasted_iota(jnp.int32, (n, n), 0)
    c = jax.lax.broadcasted_iota(jnp.int32, (n, n), 1)
    return (r == c).astype(jnp.float32)


def _transpose_pack(refs):
    r = jax.lax.broadcasted_iota(jnp.int32, (_DIM, _PDIM), 0)
    c = jax.lax.broadcasted_iota(jnp.int32, (_DIM, _PDIM), 1)
    acc = None
    for a, ref in enumerate(refs):
        sel = (r == c - _DIM * a).astype(jnp.float32)
        x = jnp.nan_to_num(ref[...], nan=0.0, posinf=0.0, neginf=0.0)
        y = jax.lax.dot_general(x, sel, (((0,), (0,)), ((), ())),
                                preferred_element_type=jnp.float32)
        acc = y if acc is None else acc + y
    return acc


def _bf16_hi_bits(x):
    return jax.lax.bitcast_convert_type(
        x.astype(jnp.bfloat16), jnp.uint16).astype(jnp.uint32)


def _pack_body(w0, w1, w2, w3, r0, r1, r2, r3, out_ref):
    wu = _transpose_pack((w0, w1, w2, w3))
    rv = _transpose_pack((r0, r1, r2, r3))
    word = (_bf16_hi_bits(wu) << 16) | _bf16_hi_bits(rv)
    out_ref[...] = jax.lax.bitcast_convert_type(word, jnp.float32)


def _tc_pack(wuT, rvhT):
    max_blk = (wuT.shape[1] - 1) // _LP
    specs = [
        pl.BlockSpec(
            (_DIM, _LP),
            (lambda a: (lambda i, a=a: (0, jnp.minimum(a * _GP + i,
                                                       max_blk))))(a))
        for a in range(_PACK)
    ] * 2
    return pl.pallas_call(
        _pack_body,
        grid=(_GP,),
        in_specs=specs,
        out_specs=pl.BlockSpec((_LP, _PDIM), lambda i: (i, 0)),
        out_shape=jax.ShapeDtypeStruct((_Q, _PDIM), jnp.float32),
    )(wuT, wuT, wuT, wuT, rvhT, rvhT, rvhT, rvhT)


def _sc_gather(idx2d, packed):
    mesh = plsc.VectorSubcoreMesh(core_axis_name="c", subcore_axis_name="s")

    @functools.partial(
        pl.kernel,
        out_type=jax.ShapeDtypeStruct((_BATCH, _PDIM), jnp.float32),
        mesh=mesh,
        scratch_types=[
            pltpu.VMEM((_NCHUNK, _CHUNK), jnp.int32),
            pltpu.VMEM((_B_PER_W, _PDIM), jnp.float32),
            pltpu.SemaphoreType.DMA,
        ],
    )
    def gather_kernel(idx_hbm, tbl_hbm, rows_out, idx_v, rows_v, sem):
        wid = lax.axis_index("s") * 2 + lax.axis_index("c")
        base = wid * _B_PER_W
        pltpu.sync_copy(idx_hbm.at[pl.ds(wid * _NCHUNK, _NCHUNK)], idx_v)
        copies = []
        for j in range(_NCHUNK):
            copies.append(pltpu.async_copy(
                tbl_hbm.at[idx_v.at[j]],
                rows_v.at[pl.ds(j * _CHUNK, _CHUNK)], sem))
        for c in copies:
            c.wait()
        pltpu.sync_copy(rows_v, rows_out.at[pl.ds(base, _B_PER_W)])

    return gather_kernel(idx2d, packed)


def _artanh(x):
    return 0.5 * jnp.log((1.0 + x) / (1.0 - x))


def _colnorm(x):
    return jnp.sqrt(jnp.sum(x * x, axis=0, keepdims=True))


def _proj_cols(e):
    n = _colnorm(e)
    return jnp.where(n >= 1.0, e / (n - 1e-05), e)


def _p_sum_cols(x, y):
    sqxnorm = jnp.sum(x * x, axis=0, keepdims=True)
    sqynorm = jnp.sum(y * y, axis=0, keepdims=True)
    dotxy = jnp.sum(x * y, axis=0, keepdims=True)
    numerator = (1.0 + 2.0 * dotxy + sqynorm) * x + (1.0 - sqxnorm) * y
    denominator = 1.0 + 2.0 * dotxy + sqxnorm * sqynorm
    return numerator / denominator


def _select_subrow(packed, sub):
    out = jnp.zeros((packed.shape[0], _DIM), packed.dtype)
    for k in range(_PACK):
        out = jnp.where(sub == k, packed[:, k * _DIM:(k + 1) * _DIM], out)
    return out


def _math_body(uT_ref, vT_ref, rows_ref, sub_ref, out_ref):
    sub = sub_ref[...]
    eye = _eye(_DIM)
    word = jax.lax.bitcast_convert_type(rows_ref[...], jnp.uint32)
    wu_f = jax.lax.bitcast_convert_type(
        word & jnp.uint32(0xFFFF0000), jnp.float32)
    rv_f = jax.lax.bitcast_convert_type(word << 16, jnp.float32)
    Ru_r = _select_subrow(wu_f, sub)
    rv_r = _select_subrow(rv_f, sub)
    Ru = jax.lax.dot_general(eye, Ru_r, (((0,), (1,)), ((), ())),
                             preferred_element_type=jnp.float32)
    rv_raw = jax.lax.dot_general(eye, rv_r, (((0,), (1,)), ((), ())),
                                 preferred_element_type=jnp.float32)
    u = _proj_cols(uT_ref[...])
    v = _proj_cols(vT_ref[...])
    rv = _proj_cols(rv_raw)
    un = jnp.clip(_colnorm(u), 1e-10, 1.0 - 1e-05)
    u_e = _artanh(un) / un * u
    u_W = u_e * Ru
    wn = jnp.maximum(_colnorm(u_W), 1e-10)
    u_m = jnp.tanh(wn) / wn * u_W
    v_m = _p_sum_cols(v, rv)
    u_m = _proj_cols(u_m)
    v_m = _proj_cols(v_m)
    diff = _p_sum_cols(-u_m, v_m)
    diff_norm = jnp.clip(_colnorm(diff), 1e-10, 1.0 - 1e-05)
    sqdist = (2.0 * _artanh(diff_norm)) ** 2
    out_ref[...] = -sqdist


def _tc_math(uT, vT, rows, sub, block_rows=2048):
    grid = _BATCH // block_rows
    t_spec = pl.BlockSpec((_DIM, block_rows), lambda i: (0, i))
    packed_spec = pl.BlockSpec((block_rows, _PDIM), lambda i: (i, 0))
    sub_spec = pl.BlockSpec((block_rows, 1), lambda i: (i, 0))
    return pl.pallas_call(
        _math_body,
        grid=(grid,),
        in_specs=[t_spec, t_spec, packed_spec, sub_spec],
        out_specs=pl.BlockSpec((1, block_rows), lambda i: (0, i)),
        out_shape=jax.ShapeDtypeStruct((1, _BATCH), jnp.float32),
    )(uT, vT, rows, sub)


def kernel(u_emb, r_idx, v_emb, Wu, rvh):
    packed = _tc_pack(Wu.T, rvh.T)
    idx2d = (r_idx % _Q).reshape(_NW * _NCHUNK, _CHUNK)
    sub = (r_idx // _Q).reshape(_BATCH, 1)
    rows = _sc_gather(idx2d, packed)
    score = _tc_math(u_emb.T, v_emb.T, rows, sub)
    return score.reshape(_BATCH)


# speedup vs baseline: 14.7220x; 1.0252x over previous
import functools

import jax
import jax.numpy as jnp
from jax import lax
from jax.experimental import pallas as pl
from jax.experimental.pallas import tpu as pltpu
from jax.experimental.pallas import tpu_sc as plsc

_BATCH = 16384
_DIM = 32
_PACK = 4
_PDIM = _DIM * _PACK
_NW = 32
_B_PER_W = _BATCH // _NW
_CHUNK = 128
_NCHUNK = _B_PER_W // _CHUNK
_LP = 8192
_GP = 31
_Q = _LP * _GP


def _eye(n):
    r = jax.lax.broadcasted_iota(jnp.int32, (n, n), 0)
    c = jax.lax.broadcasted_iota(jnp.int32, (n, n), 1)
    return (r == c).astype(jnp.float32)


def _transpose_pack(refs):
    r = jax.lax.broadcasted_iota(jnp.int32, (_DIM, _PDIM), 0)
    c = jax.lax.broadcasted_iota(jnp.int32, (_DIM, _PDIM), 1)
    acc = None
    for a, ref in enumerate(refs):
        sel = (r == c - _DIM * a).astype(jnp.float32)
        x = jnp.nan_to_num(ref[...], nan=0.0, posinf=0.0, neginf=0.0)
        y = jax.lax.dot_general(x, sel, (((0,), (0,)), ((), ())),
                                preferred_element_type=jnp.float32)
        acc = y if acc is None else acc + y
    return acc


def _bf16_hi_bits(x):
    return jax.lax.bitcast_convert_type(
        x.astype(jnp.bfloat16), jnp.uint16).astype(jnp.uint32)


def _pack_body(w0, w1, w2, w3, r0, r1, r2, r3, out_ref):
    wu = _transpose_pack((w0, w1, w2, w3))
    rv = _transpose_pack((r0, r1, r2, r3))
    word = (_bf16_hi_bits(wu) << 16) | _bf16_hi_bits(rv)
    out_ref[...] = jax.lax.bitcast_convert_type(word, jnp.float32)


def _tc_pack(wuT, rvhT):
    max_blk = (wuT.shape[1] - 1) // _LP
    specs = [
        pl.BlockSpec(
            (_DIM, _LP),
            (lambda a: (lambda i, a=a: (0, jnp.minimum(a * _GP + i,
                                                       max_blk))))(a))
        for a in range(_PACK)
    ] * 2
    return pl.pallas_call(
        _pack_body,
        grid=(_GP,),
        in_specs=specs,
        out_specs=pl.BlockSpec((_LP, _PDIM), lambda i: (i, 0)),
        out_shape=jax.ShapeDtypeStruct((_Q, _PDIM), jnp.float32),
    )(wuT, wuT, wuT, wuT, rvhT, rvhT, rvhT, rvhT)


def _sc_gather(idx2d, packed):
    mesh = plsc.VectorSubcoreMesh(core_axis_name="c", subcore_axis_name="s")

    @functools.partial(
        pl.kernel,
        out_type=jax.ShapeDtypeStruct((_BATCH, _PDIM), jnp.float32),
        mesh=mesh,
        scratch_types=[
            pltpu.VMEM((_NCHUNK, _CHUNK), jnp.int32),
            pltpu.VMEM((_B_PER_W, _PDIM), jnp.float32),
            pltpu.SemaphoreType.DMA,
        ],
    )
    def gather_kernel(idx_hbm, tbl_hbm, rows_out, idx_v, rows_v, sem):
        wid = lax.axis_index("s") * 2 + lax.axis_index("c")
        base = wid * _B_PER_W
        pltpu.sync_copy(idx_hbm.at[pl.ds(wid * _NCHUNK, _NCHUNK)], idx_v)
        copies = []
        for j in range(_NCHUNK):
            copies.append(pltpu.async_copy(
                tbl_hbm.at[idx_v.at[j]],
                rows_v.at[pl.ds(j * _CHUNK, _CHUNK)], sem))
        for c in copies:
            c.wait()
        pltpu.sync_copy(rows_v, rows_out.at[pl.ds(base, _B_PER_W)])

    return gather_kernel(idx2d, packed)


def _artanh(x):
    return 0.5 * jnp.log((1.0 + x) / (1.0 - x))


def _colnorm(x):
    return jnp.sqrt(jnp.sum(x * x, axis=0, keepdims=True))


def _proj_cols(e):
    n = _colnorm(e)
    return jnp.where(n >= 1.0, e / (n - 1e-05), e)


def _p_sum_cols(x, y):
    sqxnorm = jnp.sum(x * x, axis=0, keepdims=True)
    sqynorm = jnp.sum(y * y, axis=0, keepdims=True)
    dotxy = jnp.sum(x * y, axis=0, keepdims=True)
    numerator = (1.0 + 2.0 * dotxy + sqynorm) * x + (1.0 - sqxnorm) * y
    denominator = 1.0 + 2.0 * dotxy + sqxnorm * sqynorm
    return numerator / denominator


def _select_subrow(packed, sub):
    out = jnp.zeros((packed.shape[0], _DIM), packed.dtype)
    for k in range(_PACK):
        out = jnp.where(sub == k, packed[:, k * _DIM:(k + 1) * _DIM], out)
    return out


def _math_body(uT_ref, vT_ref, rows_ref, sub_ref, out_ref):
    sub = sub_ref[...]
    eye = _eye(_DIM)
    word = jax.lax.bitcast_convert_type(rows_ref[...], jnp.uint32)
    wu_f = jax.lax.bitcast_convert_type(
        word & jnp.uint32(0xFFFF0000), jnp.float32)
    rv_f = jax.lax.bitcast_convert_type(word << 16, jnp.float32)
    Ru_r = _select_subrow(wu_f, sub)
    rv_r = _select_subrow(rv_f, sub)
    Ru = jax.lax.dot_general(eye, Ru_r, (((0,), (1,)), ((), ())),
                             preferred_element_type=jnp.float32)
    rv_raw = jax.lax.dot_general(eye, rv_r, (((0,), (1,)), ((), ())),
                                 preferred_element_type=jnp.float32)
    u = _proj_cols(uT_ref[...])
    v = _proj_cols(vT_ref[...])
    rv = _proj_cols(rv_raw)
    un = jnp.clip(_colnorm(u), 1e-10, 1.0 - 1e-05)
    u_e = _artanh(un) / un * u
    u_W = u_e * Ru
    wn = jnp.maximum(_colnorm(u_W), 1e-10)
    u_m = jnp.tanh(wn) / wn * u_W
    v_m = _p_sum_cols(v, rv)
    u_m = _proj_cols(u_m)
    v_m = _proj_cols(v_m)
    diff = _p_sum_cols(-u_m, v_m)
    diff_norm = jnp.clip(_colnorm(diff), 1e-10, 1.0 - 1e-05)
    sqdist = (2.0 * _artanh(diff_norm)) ** 2
    out_ref[...] = -sqdist


def _tc_math(uT, vT, rows, sub, block_rows=2048):
    grid = _BATCH // block_rows
    t_spec = pl.BlockSpec((_DIM, block_rows), lambda i: (0, i))
    packed_spec = pl.BlockSpec((block_rows, _PDIM), lambda i: (i, 0))
    sub_spec = pl.BlockSpec((block_rows, 1), lambda i: (i, 0))
    return pl.pallas_call(
        _math_body,
        grid=(grid,),
        in_specs=[t_spec, t_spec, packed_spec, sub_spec],
        out_specs=pl.BlockSpec((1, block_rows), lambda i: (0, i)),
        out_shape=jax.ShapeDtypeStruct((1, _BATCH), jnp.float32),
    )(uT, vT, rows, sub)


def kernel(u_emb, r_idx, v_emb, Wu, rvh):
    packed = _tc_pack(Wu.T, rvh.T)
    idx2d = (r_idx % _Q).reshape(_NW * _NCHUNK, _CHUNK)
    sub = (r_idx // _Q).reshape(_BATCH, 1)
    rows = _sc_gather(idx2d, packed)
    score = _tc_math(u_emb.T, v_emb.T, rows, sub)
    return score.reshape(_BATCH)
